# Initial kernel scaffold; baseline (speedup 1.0000x reference)
import jax
import jax.numpy as jnp
from jax.experimental import pallas as pl


def kernel(continuous_inputs, pitcher_id, pitch_type, batter_side, game_situation, W_cont, b_cont, E_pitcher_id, E_pitch_type, E_batter_side, E_game_situation, W_final, b_final):
    raise NotImplementedError("write your pallas kernel here")


# speedup vs baseline: 1.6181x; 1.6181x over previous
import functools

import jax
import jax.numpy as jnp
from jax import lax
from jax.experimental import pallas as pl
from jax.experimental.pallas import tpu as pltpu
from jax.experimental.pallas import tpu_sc as plsc

B = 16384
CONT_DIM = 256
OUT_DIM = 256
HALF = 128
D1, D2, D3, D4 = 64, 32, 16, 32

NC, NS = 2, 16
NW = NC * NS
BPW = B // NW


def _sc_gather(pid, pt, bs, gs, E1, E2, E3, E4):
    mesh = plsc.VectorSubcoreMesh(core_axis_name="c", subcore_axis_name="s")
    f32 = jnp.float32

    @functools.partial(
        pl.kernel,
        out_type=(
            jax.ShapeDtypeStruct((B, D1), f32),
            jax.ShapeDtypeStruct((B, D2), f32),
            jax.ShapeDtypeStruct((B, D3), f32),
            jax.ShapeDtypeStruct((B, D4), f32),
        ),
        mesh=mesh,
        compiler_params=pltpu.CompilerParams(use_tc_tiling_on_sc=False),
        scratch_types=[
            pltpu.VMEM((BPW,), jnp.int32),
            pltpu.VMEM((BPW,), jnp.int32),
            pltpu.VMEM((BPW,), jnp.int32),
            pltpu.VMEM((BPW,), jnp.int32),
            pltpu.VMEM((BPW, D1), f32),
            pltpu.VMEM((BPW, D2), f32),
            pltpu.VMEM((BPW, D3), f32),
            pltpu.VMEM((BPW, D4), f32),
            pltpu.SemaphoreType.DMA,
            pltpu.SemaphoreType.DMA,
            pltpu.SemaphoreType.DMA,
            pltpu.SemaphoreType.DMA,
        ],
    )
    def k(pid_h, pt_h, bs_h, gs_h, e1_h, e2_h, e3_h, e4_h,
          o1_h, o2_h, o3_h, o4_h,
          i1, i2, i3, i4, r1, r2, r3, r4, s1, s2, s3, s4):
        wid = lax.axis_index("s") * NC + lax.axis_index("c")
        base = wid * BPW
        sl = pl.ds(base, BPW)
        pltpu.sync_copy(pid_h.at[sl], i1)
        pltpu.sync_copy(pt_h.at[sl], i2)
        pltpu.sync_copy(bs_h.at[sl], i3)
        pltpu.sync_copy(gs_h.at[sl], i4)
        c1 = pltpu.async_copy(e1_h.at[i1], r1, s1)
        c2 = pltpu.async_copy(e2_h.at[i2], r2, s2)
        c3 = pltpu.async_copy(e3_h.at[i3], r3, s3)
        c4 = pltpu.async_copy(e4_h.at[i4], r4, s4)
        c1.wait()
        pltpu.sync_copy(r1, o1_h.at[sl])
        c2.wait()
        pltpu.sync_copy(r2, o2_h.at[sl])
        c3.wait()
        pltpu.sync_copy(r3, o3_h.at[sl])
        c4.wait()
        pltpu.sync_copy(r4, o4_h.at[sl])

    return k(pid, pt, bs, gs, E1, E2, E3, E4)


def _tc_body(x_ref, g1_ref, g2_ref, g3_ref, g4_ref,
             wc_ref, bc_ref, w0_ref, w1_ref, w2_ref, w3_ref, w4_ref,
             bf_ref, out_ref):
    f32 = jnp.float32
    cont = jnp.dot(x_ref[...], wc_ref[...], preferred_element_type=f32)
    cont = cont + bc_ref[...]
    acc = jnp.dot(cont, w0_ref[...], preferred_element_type=f32)
    acc = acc + jnp.dot(g1_ref[...], w1_ref[...], preferred_element_type=f32)
    acc = acc + jnp.dot(g2_ref[...], w2_ref[...], preferred_element_type=f32)
    acc = acc + jnp.dot(g3_ref[...], w3_ref[...], preferred_element_type=f32)
    acc = acc + jnp.dot(g4_ref[...], w4_ref[...], preferred_element_type=f32)
    out_ref[...] = acc + bf_ref[...]


def kernel(continuous_inputs, pitcher_id, pitch_type, batter_side,
           game_situation, W_cont, b_cont, E_pitcher_id, E_pitch_type,
           E_batter_side, E_game_situation, W_final, b_final):
    i32 = jnp.int32
    g1, g2, g3, g4 = _sc_gather(
        pitcher_id.astype(i32), pitch_type.astype(i32),
        batter_side.astype(i32), game_situation.astype(i32),
        E_pitcher_id, E_pitch_type, E_batter_side, E_game_situation)

    w0 = W_final[:HALF]
    w1 = W_final[HALF:HALF + D1]
    w2 = W_final[HALF + D1:HALF + D1 + D2]
    w3 = W_final[HALF + D1 + D2:HALF + D1 + D2 + D3]
    w4 = W_final[HALF + D1 + D2 + D3:]
    bc = b_cont.reshape(1, HALF)
    bf = b_final.reshape(1, OUT_DIM)

    BM = 1024
    grid = (B // BM,)
    row = lambda i: (i, 0)
    full = lambda i: (0, 0)
    out = pl.pallas_call(
        _tc_body,
        grid=grid,
        in_specs=[
            pl.BlockSpec((BM, CONT_DIM), row),
            pl.BlockSpec((BM, D1), row),
            pl.BlockSpec((BM, D2), row),
            pl.BlockSpec((BM, D3), row),
            pl.BlockSpec((BM, D4), row),
            pl.BlockSpec((CONT_DIM, HALF), full),
            pl.BlockSpec((1, HALF), full),
            pl.BlockSpec((HALF, OUT_DIM), full),
            pl.BlockSpec((D1, OUT_DIM), full),
            pl.BlockSpec((D2, OUT_DIM), full),
            pl.BlockSpec((D3, OUT_DIM), full),
            pl.BlockSpec((D4, OUT_DIM), full),
            pl.BlockSpec((1, OUT_DIM), full),
        ],
        out_specs=pl.BlockSpec((BM, OUT_DIM), row),
        out_shape=jax.ShapeDtypeStruct((B, OUT_DIM), jnp.float32),
        compiler_params=pltpu.CompilerParams(
            dimension_semantics=("arbitrary",),
        ),
    )(continuous_inputs, g1, g2, g3, g4, W_cont, bc, w0, w1, w2, w3, w4, bf)
    return out


# speedup vs baseline: 2.4222x; 1.4970x over previous
import functools

import jax
import jax.numpy as jnp
from jax import lax
from jax.experimental import pallas as pl
from jax.experimental.pallas import tpu as pltpu
from jax.experimental.pallas import tpu_sc as plsc

B = 16384
CONT_DIM = 256
OUT_DIM = 256
HALF = 128
D1, D2, D3, D4 = 64, 32, 16, 32

NC, NS = 2, 16
NW = NC * NS
BPW = B // NW


def _sc_gather(idx3, E1, E2, E4):
    mesh = plsc.VectorSubcoreMesh(core_axis_name="c", subcore_axis_name="s")
    f32 = jnp.float32

    @functools.partial(
        pl.kernel,
        out_type=jax.ShapeDtypeStruct((B, 128), f32),
        mesh=mesh,
        compiler_params=pltpu.CompilerParams(use_tc_tiling_on_sc=False),
        scratch_types=[
            pltpu.VMEM((3, BPW), jnp.int32),
            pltpu.VMEM((BPW, D1), f32),
            pltpu.VMEM((BPW, D2), f32),
            pltpu.VMEM((BPW, D4), f32),
            pltpu.SemaphoreType.DMA,
            pltpu.SemaphoreType.DMA,
            pltpu.SemaphoreType.DMA,
            pltpu.SemaphoreType.DMA,
            pltpu.SemaphoreType.DMA,
        ],
    )
    def k(idx_h, e1_h, e2_h, e4_h, o_h, idxv, r1, r2, r4, si, s1, s2, s4, sw):
        wid = lax.axis_index("s") * NC + lax.axis_index("c")
        base = wid * BPW
        sl = pl.ds(base, BPW)
        pltpu.async_copy(idx_h.at[:, sl], idxv, si).wait()
        g1 = pltpu.async_copy(e1_h.at[idxv.at[0]], r1, s1)
        g2 = pltpu.async_copy(e2_h.at[idxv.at[1]], r2, s2)
        g4 = pltpu.async_copy(e4_h.at[idxv.at[2]], r4, s4)
        g2.wait()
        w2 = pltpu.async_copy(r2, o_h.at[sl, pl.ds(D1, D2)], sw)
        g4.wait()
        w4 = pltpu.async_copy(r4, o_h.at[sl, pl.ds(D1 + D2, D4)], sw)
        g1.wait()
        w1 = pltpu.async_copy(r1, o_h.at[sl, pl.ds(0, D1)], sw)
        w2.wait()
        w4.wait()
        w1.wait()

    return k(idx3, E1, E2, E4)


def _tc_body(x_ref, comb_ref, bs_ref, wc_ref, bc_ref, w0_ref, wcat_ref,
             e3_ref, w3_ref, bf_ref, out_ref):
    f32 = jnp.float32
    cont = jnp.dot(x_ref[...], wc_ref[...], preferred_element_type=f32)
    cont = cont + bc_ref[...]
    acc = jnp.dot(cont, w0_ref[...], preferred_element_type=f32)
    acc = acc + jnp.dot(comb_ref[...], wcat_ref[...],
                        preferred_element_type=f32)
    t3 = jnp.dot(e3_ref[...], w3_ref[...], preferred_element_type=f32)
    onehot = (lax.broadcasted_iota(jnp.int32, (x_ref.shape[0], D3), 1)
              == bs_ref[...]).astype(f32)
    acc = acc + jnp.dot(onehot, t3, preferred_element_type=f32)
    out_ref[...] = acc + bf_ref[...]


def kernel(continuous_inputs, pitcher_id, pitch_type, batter_side,
           game_situation, W_cont, b_cont, E_pitcher_id, E_pitch_type,
           E_batter_side, E_game_situation, W_final, b_final):
    i32 = jnp.int32
    idx3 = jnp.stack([pitcher_id.astype(i32), pitch_type.astype(i32),
                      game_situation.astype(i32)])
    comb = _sc_gather(idx3, E_pitcher_id, E_pitch_type, E_game_situation)

    w0 = W_final[:HALF]
    wcat = jnp.concatenate(
        [W_final[HALF:HALF + D1 + D2], W_final[HALF + D1 + D2 + D3:]], axis=0)
    w3 = W_final[HALF + D1 + D2:HALF + D1 + D2 + D3]
    bc = b_cont.reshape(1, HALF)
    bf = b_final.reshape(1, OUT_DIM)
    bs2 = batter_side.astype(i32).reshape(B, 1)

    BM = 1024
    grid = (B // BM,)
    row = lambda i: (i, 0)
    full = lambda i: (0, 0)
    out = pl.pallas_call(
        _tc_body,
        grid=grid,
        in_specs=[
            pl.BlockSpec((BM, CONT_DIM), row),
            pl.BlockSpec((BM, 128), row),
            pl.BlockSpec((BM, 1), row),
            pl.BlockSpec((CONT_DIM, HALF), full),
            pl.BlockSpec((1, HALF), full),
            pl.BlockSpec((HALF, OUT_DIM), full),
            pl.BlockSpec((128, OUT_DIM), full),
            pl.BlockSpec((D3, D3), full),
            pl.BlockSpec((D3, OUT_DIM), full),
            pl.BlockSpec((1, OUT_DIM), full),
        ],
        out_specs=pl.BlockSpec((BM, OUT_DIM), row),
        out_shape=jax.ShapeDtypeStruct((B, OUT_DIM), jnp.float32),
        compiler_params=pltpu.CompilerParams(
            dimension_semantics=("arbitrary",),
        ),
    )(continuous_inputs, comb, bs2, W_cont, bc, w0, wcat, E_batter_side,
      w3, bf)
    return out


# speedup vs baseline: 2.5675x; 1.0600x over previous
import functools

import jax
import jax.numpy as jnp
from jax import lax
from jax.experimental import pallas as pl
from jax.experimental.pallas import tpu as pltpu
from jax.experimental.pallas import tpu_sc as plsc

B = 16384
CONT_DIM = 256
OUT_DIM = 256
HALF = 128
D1, D2, D3, D4 = 64, 32, 16, 32

NC, NS = 2, 16
NW = NC * NS
BPW = B // NW


def _sc_gather(idx3, E1p, E2, E4):
    mesh = plsc.VectorSubcoreMesh(core_axis_name="c", subcore_axis_name="s")
    f32 = jnp.float32

    @functools.partial(
        pl.kernel,
        out_type=(
            jax.ShapeDtypeStruct((B, 128), f32),
            jax.ShapeDtypeStruct((B, 128), f32),
        ),
        mesh=mesh,
        compiler_params=pltpu.CompilerParams(use_tc_tiling_on_sc=False),
        scratch_types=[
            pltpu.VMEM((3, BPW), jnp.int32),
            pltpu.VMEM((BPW, 128), f32),
            pltpu.VMEM((BPW, D2), f32),
            pltpu.VMEM((BPW, D4), f32),
            pltpu.SemaphoreType.DMA,
            pltpu.SemaphoreType.DMA,
            pltpu.SemaphoreType.DMA,
            pltpu.SemaphoreType.DMA,
            pltpu.SemaphoreType.DMA,
        ],
    )
    def k(idx_h, e1_h, e2_h, e4_h, o1_h, oc_h,
          idxv, r1, r2, r4, si, s1, s2, s4, sw):
        wid = lax.axis_index("s") * NC + lax.axis_index("c")
        base = wid * BPW
        sl = pl.ds(base, BPW)
        pltpu.async_copy(idx_h.at[:, sl], idxv, si).wait()
        g1 = pltpu.async_copy(e1_h.at[idxv.at[0]], r1, s1)
        g2 = pltpu.async_copy(e2_h.at[idxv.at[1]], r2, s2)
        g4 = pltpu.async_copy(e4_h.at[idxv.at[2]], r4, s4)
        g2.wait()
        w2 = pltpu.async_copy(r2, oc_h.at[sl, pl.ds(0, D2)], sw)
        g4.wait()
        w4 = pltpu.async_copy(r4, oc_h.at[sl, pl.ds(D2, D4)], sw)
        g1.wait()
        w1 = pltpu.async_copy(r1, o1_h.at[sl], sw)
        w2.wait()
        w4.wait()
        w1.wait()

    return k(idx3, E1p, E2, E4)


def _tc_body(x_ref, o1_ref, comb_ref, aux_ref, wc_ref, bc_ref,
             w0_ref, w1s_ref, wcat_ref, e3_ref, w3_ref, bf_ref, out_ref):
    f32 = jnp.float32
    i32 = jnp.int32
    bm = x_ref.shape[0]

    blk = aux_ref[...]
    r8 = lax.broadcasted_iota(i32, (bm, 8), 0) // 128
    oh8 = (r8 == lax.broadcasted_iota(i32, (bm, 8), 1)).astype(f32)
    rows = jnp.dot(oh8, blk, preferred_element_type=f32)
    lsel = (lax.broadcasted_iota(i32, (bm, 128), 0) % 128
            == lax.broadcasted_iota(i32, (bm, 128), 1)).astype(f32)
    bsv = jnp.sum(rows * lsel, axis=1, keepdims=True).astype(i32)

    cont = jnp.dot(x_ref[...], wc_ref[...], preferred_element_type=f32)
    cont = cont + bc_ref[...]
    acc = jnp.dot(cont, w0_ref[...], preferred_element_type=f32)

    acc = acc + jnp.dot(o1_ref[...], w1s_ref[...],
                        preferred_element_type=f32)
    lt64 = lax.broadcasted_iota(i32, (bm, 128), 1) < 64
    combz = jnp.where(lt64, comb_ref[...], 0.0)
    acc = acc + jnp.dot(combz, wcat_ref[...], preferred_element_type=f32)

    t3 = jnp.dot(e3_ref[...], w3_ref[...], preferred_element_type=f32)
    onehot = (lax.broadcasted_iota(i32, (bm, D3), 1) == bsv).astype(f32)
    acc = acc + jnp.dot(onehot, t3, preferred_element_type=f32)
    out_ref[...] = acc + bf_ref[...]


def kernel(continuous_inputs, pitcher_id, pitch_type, batter_side,
           game_situation, W_cont, b_cont, E_pitcher_id, E_pitch_type,
           E_batter_side, E_game_situation, W_final, b_final):
    i32 = jnp.int32
    pid = pitcher_id.astype(i32)
    idx3 = jnp.stack([pid, pitch_type.astype(i32),
                      game_situation.astype(i32)])
    E1p = jnp.pad(E_pitcher_id, ((0, 0), (0, 128 - D1)))
    o1p, comb = _sc_gather(idx3, E1p, E_pitch_type, E_game_situation)

    aux2d = batter_side.astype(jnp.float32).reshape(128, 128)

    w0 = W_final[:HALF]
    w1 = W_final[HALF:HALF + D1]
    w1s = jnp.concatenate(
        [w1, jnp.zeros((128 - D1, OUT_DIM), jnp.float32)], axis=0)
    wcat = jnp.concatenate(
        [W_final[HALF + D1:HALF + D1 + D2], W_final[HALF + D1 + D2 + D3:],
         jnp.zeros((128 - D2 - D4, OUT_DIM), jnp.float32)],
        axis=0)
    w3 = W_final[HALF + D1 + D2:HALF + D1 + D2 + D3]
    bc = b_cont.reshape(1, HALF)
    bf = b_final.reshape(1, OUT_DIM)

    BM = 1024
    grid = (B // BM,)
    row = lambda i: (i, 0)
    full = lambda i: (0, 0)
    out = pl.pallas_call(
        _tc_body,
        grid=grid,
        in_specs=[
            pl.BlockSpec((BM, CONT_DIM), row),
            pl.BlockSpec((BM, 128), row),
            pl.BlockSpec((BM, 128), row),
            pl.BlockSpec((8, 128), row),
            pl.BlockSpec((CONT_DIM, HALF), full),
            pl.BlockSpec((1, HALF), full),
            pl.BlockSpec((HALF, OUT_DIM), full),
            pl.BlockSpec((128, OUT_DIM), full),
            pl.BlockSpec((128, OUT_DIM), full),
            pl.BlockSpec((D3, D3), full),
            pl.BlockSpec((D3, OUT_DIM), full),
            pl.BlockSpec((1, OUT_DIM), full),
        ],
        out_specs=pl.BlockSpec((BM, OUT_DIM), row),
        out_shape=jax.ShapeDtypeStruct((B, OUT_DIM), jnp.float32),
        compiler_params=pltpu.CompilerParams(
            dimension_semantics=("arbitrary",),
        ),
    )(continuous_inputs, o1p, comb, aux2d, W_cont, bc, w0, w1s, wcat,
      E_batter_side, w3, bf)
    return out


# speedup vs baseline: 2.6931x; 1.0489x over previous
import functools

import jax
import jax.numpy as jnp
from jax import lax
from jax.experimental import pallas as pl
from jax.experimental.pallas import tpu as pltpu
from jax.experimental.pallas import tpu_sc as plsc

B = 16384
CONT_DIM = 256
OUT_DIM = 256
HALF = 128
D1, D2, D3, D4 = 64, 32, 16, 32

NC, NS = 2, 16
NW = NC * NS
BPW = B // NW


def _sc_gather(idx3, E1p, E2, E4):
    mesh = plsc.VectorSubcoreMesh(core_axis_name="c", subcore_axis_name="s")
    f32 = jnp.float32

    @functools.partial(
        pl.kernel,
        out_type=(
            jax.ShapeDtypeStruct((B, 128), f32),
            jax.ShapeDtypeStruct((B, 128), f32),
        ),
        mesh=mesh,
        compiler_params=pltpu.CompilerParams(use_tc_tiling_on_sc=False),
        scratch_types=[
            pltpu.VMEM((3, BPW), jnp.int32),
            pltpu.VMEM((BPW, 128), f32),
            pltpu.VMEM((BPW, D2), f32),
            pltpu.VMEM((BPW, D4), f32),
            pltpu.SemaphoreType.DMA,
            pltpu.SemaphoreType.DMA,
            pltpu.SemaphoreType.DMA,
            pltpu.SemaphoreType.DMA,
            pltpu.SemaphoreType.DMA,
        ],
    )
    def k(idx_h, e1_h, e2_h, e4_h, o1_h, oc_h,
          idxv, r1, r2, r4, si, s1, s2, s4, sw):
        wid = lax.axis_index("s") * NC + lax.axis_index("c")
        base = wid * BPW
        sl = pl.ds(base, BPW)
        pltpu.async_copy(idx_h.at[:, sl], idxv, si).wait()
        g1 = pltpu.async_copy(e1_h.at[idxv.at[0]], r1, s1)
        g2 = pltpu.async_copy(e2_h.at[idxv.at[1]], r2, s2)
        g4 = pltpu.async_copy(e4_h.at[idxv.at[2]], r4, s4)
        g2.wait()
        w2 = pltpu.async_copy(r2, oc_h.at[sl, pl.ds(0, D2)], sw)
        g4.wait()
        w4 = pltpu.async_copy(r4, oc_h.at[sl, pl.ds(D2, D4)], sw)
        g1.wait()
        w1 = pltpu.async_copy(r1, o1_h.at[sl], sw)
        w2.wait()
        w4.wait()
        w1.wait()

    return k(idx3, E1p, E2, E4)


def _tc_body(x_ref, o1_ref, comb_ref, aux_ref, wc_ref, bc_ref,
             w0_ref, w1s_ref, wcat_ref, e3_ref, w3_ref, bf_ref, out_ref):
    f32 = jnp.float32
    i32 = jnp.int32
    bf16 = jnp.bfloat16
    bm = x_ref.shape[0]

    blk = aux_ref[...]
    r8 = lax.broadcasted_iota(i32, (bm, 8), 0) // 128
    oh8 = (r8 == lax.broadcasted_iota(i32, (bm, 8), 1)).astype(f32)
    rows = jnp.dot(oh8, blk, preferred_element_type=f32)
    lsel = (lax.broadcasted_iota(i32, (bm, 128), 0) % 128
            == lax.broadcasted_iota(i32, (bm, 128), 1)).astype(f32)
    bsv = jnp.sum(rows * lsel, axis=1, keepdims=True).astype(i32)

    cont = jnp.dot(x_ref[...].astype(bf16), wc_ref[...],
                   preferred_element_type=f32)
    cont = cont + bc_ref[...]
    acc = jnp.dot(cont.astype(bf16), w0_ref[...], preferred_element_type=f32)

    acc = acc + jnp.dot(o1_ref[...].astype(bf16), w1s_ref[...],
                        preferred_element_type=f32)
    lt64 = lax.broadcasted_iota(i32, (bm, 128), 1) < 64
    combz = jnp.where(lt64, comb_ref[...], 0.0).astype(bf16)
    acc = acc + jnp.dot(combz, wcat_ref[...], preferred_element_type=f32)

    t3 = jnp.dot(e3_ref[...], w3_ref[...], preferred_element_type=f32)
    onehot = (lax.broadcasted_iota(i32, (bm, D3), 1) == bsv).astype(bf16)
    acc = acc + jnp.dot(onehot, t3.astype(bf16), preferred_element_type=f32)
    out_ref[...] = acc + bf_ref[...]


def kernel(continuous_inputs, pitcher_id, pitch_type, batter_side,
           game_situation, W_cont, b_cont, E_pitcher_id, E_pitch_type,
           E_batter_side, E_game_situation, W_final, b_final):
    i32 = jnp.int32
    pid = pitcher_id.astype(i32)
    idx3 = jnp.stack([pid, pitch_type.astype(i32),
                      game_situation.astype(i32)])
    eyepad = jnp.eye(D1, 128, dtype=jnp.float32)
    E1p = jnp.dot(E_pitcher_id, eyepad, precision=jax.lax.Precision.HIGHEST)
    o1p, comb = _sc_gather(idx3, E1p, E_pitch_type, E_game_situation)

    aux2d = batter_side.astype(jnp.float32).reshape(128, 128)

    bf16 = jnp.bfloat16
    w0 = W_final[:HALF].astype(bf16)
    w1 = W_final[HALF:HALF + D1]
    w1s = jnp.concatenate(
        [w1, jnp.zeros((128 - D1, OUT_DIM), jnp.float32)], axis=0).astype(bf16)
    wcat = jnp.concatenate(
        [W_final[HALF + D1:HALF + D1 + D2], W_final[HALF + D1 + D2 + D3:],
         jnp.zeros((128 - D2 - D4, OUT_DIM), jnp.float32)],
        axis=0).astype(bf16)
    w3 = W_final[HALF + D1 + D2:HALF + D1 + D2 + D3]
    bc = b_cont.reshape(1, HALF)
    bf = b_final.reshape(1, OUT_DIM)

    BM = 1024
    grid = (B // BM,)
    row = lambda i: (i, 0)
    full = lambda i: (0, 0)
    out = pl.pallas_call(
        _tc_body,
        grid=grid,
        in_specs=[
            pl.BlockSpec((BM, CONT_DIM), row),
            pl.BlockSpec((BM, 128), row),
            pl.BlockSpec((BM, 128), row),
            pl.BlockSpec((8, 128), row),
            pl.BlockSpec((CONT_DIM, HALF), full),
            pl.BlockSpec((1, HALF), full),
            pl.BlockSpec((HALF, OUT_DIM), full),
            pl.BlockSpec((128, OUT_DIM), full),
            pl.BlockSpec((128, OUT_DIM), full),
            pl.BlockSpec((D3, D3), full),
            pl.BlockSpec((D3, OUT_DIM), full),
            pl.BlockSpec((1, OUT_DIM), full),
        ],
        out_specs=pl.BlockSpec((BM, OUT_DIM), row),
        out_shape=jax.ShapeDtypeStruct((B, OUT_DIM), jnp.float32),
        compiler_params=pltpu.CompilerParams(
            dimension_semantics=("arbitrary",),
        ),
    )(continuous_inputs, o1p, comb, aux2d, W_cont.astype(bf16), bc, w0,
      w1s, wcat, E_batter_side, w3, bf)
    return out
